# Initial kernel scaffold; baseline (speedup 1.0000x reference)
import jax
import jax.numpy as jnp
from jax.experimental import pallas as pl


def kernel(pred, gt_boxes, gt_labels, anchors):
    raise NotImplementedError("write your pallas kernel here")


# speedup vs baseline: 12.6549x; 12.6549x over previous
import jax
import jax.numpy as jnp
from jax.experimental import pallas as pl
from jax.experimental.pallas import tpu as pltpu

B, G, NC = 8, 20, 80
A, H, W = 3, 80, 80
PER = 5 + NC
S = H * W
SR, SL = 10, 640
N = S * A
POS_T, NEG_T, RATIO = 0.5, 0.3, 3
W_OBJ, W_CLS, W_LOC = 1.0, 1.0, 2.0


def _loss_kernel(pred_ref, anc_ref, gt_ref, lab_ref, out_ref, accf, acci):
    b = pl.program_id(0)

    @pl.when(b == 0)
    def _init():
        accf[0] = 0.0
        accf[1] = 0.0
        accf[2] = 0.0
        acci[0] = 0
        acci[1] = 0

    pred_b = pred_ref[0]

    sum_obj_pos = jnp.float32(0.0)
    sum_cls = jnp.float32(0.0)
    sum_loc = jnp.float32(0.0)
    num_pos = jnp.int32(0)
    num_neg = jnp.int32(0)
    negv_list = []

    for a in range(A):
        ax1 = anc_ref[a, 0]
        ay1 = anc_ref[a, 1]
        ax2 = anc_ref[a, 2]
        ay2 = anc_ref[a, 3]
        area_a = (ax2 - ax1) * (ay2 - ay1)

        best = jnp.full((SR, SL), -1.0, jnp.float32)
        mlab = jnp.zeros((SR, SL), jnp.int32)
        bx1 = jnp.zeros((SR, SL), jnp.float32)
        by1 = jnp.zeros((SR, SL), jnp.float32)
        bx2 = jnp.zeros((SR, SL), jnp.float32)
        by2 = jnp.zeros((SR, SL), jnp.float32)
        for g in range(G):
            gx1 = gt_ref[b, g, 0]
            gy1 = gt_ref[b, g, 1]
            gx2 = gt_ref[b, g, 2]
            gy2 = gt_ref[b, g, 3]
            gl = lab_ref[b, g]
            x1 = jnp.maximum(ax1, gx1)
            y1 = jnp.maximum(ay1, gy1)
            x2 = jnp.minimum(ax2, gx2)
            y2 = jnp.minimum(ay2, gy2)
            inter = jnp.clip(x2 - x1, 0.0, None) * jnp.clip(y2 - y1, 0.0, None)
            ag = (gx2 - gx1) * (gy2 - gy1)
            iou = inter / jnp.maximum(area_a + ag - inter, 1e-9)
            upd = iou > best
            best = jnp.where(upd, iou, best)
            mlab = jnp.where(upd, gl, mlab)
            bx1 = jnp.where(upd, gx1, bx1)
            by1 = jnp.where(upd, gy1, by1)
            bx2 = jnp.where(upd, gx2, bx2)
            by2 = jnp.where(upd, gy2, by2)

        pos = best >= POS_T
        neg = best < NEG_T
        posf = pos.astype(jnp.float32)
        num_pos = num_pos + jnp.sum(pos.astype(jnp.int32))
        num_neg = num_neg + jnp.sum(neg.astype(jnp.int32))

        x = pred_b[a * PER + 4]
        bce = jnp.clip(x, 0.0, None) - x * posf + jnp.log1p(jnp.exp(-jnp.abs(x)))
        sum_obj_pos = sum_obj_pos + jnp.sum(jnp.where(pos, bce, 0.0))
        negv_list.append(jnp.where(neg, bce, -1.0))

        logits = pred_b[a * PER + 5: a * PER + 5 + NC]
        m = jnp.max(logits, axis=0)
        lse = m + jnp.log(jnp.sum(jnp.exp(logits - m[None]), axis=0))
        cidx = jax.lax.broadcasted_iota(jnp.int32, (NC, SR, SL), 0)
        picked = jnp.sum(jnp.where(cidx == mlab[None], logits, 0.0), axis=0)
        sum_cls = sum_cls + jnp.sum(jnp.where(pos, lse - picked, 0.0))

        aw = jnp.clip(ax2 - ax1, 1e-6, None)
        ah = jnp.clip(ay2 - ay1, 1e-6, None)
        acx = (ax1 + ax2) * 0.5
        acy = (ay1 + ay2) * 0.5
        gw = jnp.clip(bx2 - bx1, 1e-6, None)
        gh = jnp.clip(by2 - by1, 1e-6, None)
        gcx = (bx1 + bx2) * 0.5
        gcy = (by1 + by2) * 0.5
        tgts = [(gcx - acx) / aw, (gcy - acy) / ah,
                jnp.log(gw / aw), jnp.log(gh / ah)]
        loc_acc = jnp.zeros((SR, SL), jnp.float32)
        for c in range(4):
            d = pred_b[a * PER + c] - tgts[c]
            ad = jnp.abs(d)
            sl = jnp.where(ad < 1.0, 0.5 * d * d, ad - 0.5)
            loc_acc = loc_acc + sl
        sum_loc = sum_loc + jnp.sum(jnp.where(pos, loc_acc, 0.0))

    negv = jnp.stack(negv_list, axis=0)
    k = jnp.where(num_pos > 0, RATIO * num_pos, jnp.minimum(num_neg, 100))
    k = jnp.minimum(k, num_neg)

    iv = jax.lax.bitcast_convert_type(negv, jnp.int32)

    def bs_body(_, carry):
        lo, hi = carry
        mid = lo + ((hi - lo + 1) >> 1)
        cnt = jnp.sum((iv >= mid).astype(jnp.int32))
        cond = cnt >= k
        return (jnp.where(cond, mid, lo), jnp.where(cond, hi, mid - 1))

    lo_bits, _ = jax.lax.fori_loop(
        0, 31, bs_body, (jnp.int32(0), jnp.int32(0x7F800000)))
    tval = jnp.max(jnp.where(iv == lo_bits, negv, 0.0))
    cnt_gt = jnp.sum((iv > lo_bits).astype(jnp.int32))
    sum_gt = jnp.sum(jnp.where(iv > lo_bits, negv, 0.0))
    topk = sum_gt + (k - cnt_gt).astype(jnp.float32) * tval
    topk = jnp.where(k > 0, topk, 0.0)

    accf[0] = accf[0] + sum_obj_pos + topk
    accf[1] = accf[1] + sum_cls
    accf[2] = accf[2] + sum_loc
    acci[0] = acci[0] + num_pos
    acci[1] = acci[1] + num_pos + k

    @pl.when(b == B - 1)
    def _final():
        dp = jnp.maximum(acci[0], 1).astype(jnp.float32)
        do = jnp.maximum(acci[1], 1).astype(jnp.float32)
        lo_l = accf[0] / do * W_OBJ
        lc_l = accf[1] / dp * W_CLS
        ll_l = accf[2] / dp * W_LOC
        out_ref[0] = lo_l
        out_ref[1] = lc_l
        out_ref[2] = ll_l
        out_ref[3] = lo_l + lc_l + ll_l


@jax.jit
def kernel(pred, gt_boxes, gt_labels, anchors):
    pred_r = pred.reshape(B, A * PER, SR, SL)
    anc_r = jnp.transpose(anchors.reshape(S, A, 4), (1, 2, 0)).reshape(A, 4, SR, SL)
    gt = gt_boxes.astype(jnp.float32)
    lab = gt_labels.astype(jnp.int32)

    out = pl.pallas_call(
        _loss_kernel,
        grid=(B,),
        in_specs=[
            pl.BlockSpec((1, A * PER, SR, SL), lambda b: (b, 0, 0, 0)),
            pl.BlockSpec((A, 4, SR, SL), lambda b: (0, 0, 0, 0)),
            pl.BlockSpec(memory_space=pltpu.SMEM),
            pl.BlockSpec(memory_space=pltpu.SMEM),
        ],
        out_specs=pl.BlockSpec(memory_space=pltpu.SMEM),
        out_shape=jax.ShapeDtypeStruct((4,), jnp.float32),
        scratch_shapes=[
            pltpu.SMEM((4,), jnp.float32),
            pltpu.SMEM((4,), jnp.int32),
        ],
    )(pred_r, anc_r, gt, lab)
    return out


# speedup vs baseline: 13.5430x; 1.0702x over previous
import jax
import jax.numpy as jnp
from jax.experimental import pallas as pl
from jax.experimental.pallas import tpu as pltpu

B, G, NC = 8, 20, 80
A, H, W = 3, 80, 80
PER = 5 + NC
S = H * W
SR, SL = 10, 640
N = S * A
POS_T, NEG_T, RATIO = 0.5, 0.3, 3
W_OBJ, W_CLS, W_LOC = 1.0, 1.0, 2.0


def _loss_kernel(pred_ref, anc_ref, gt_ref, lab_ref, out_ref, accf, acci):
    b = pl.program_id(0)

    @pl.when(b == 0)
    def _init():
        accf[0] = 0.0
        accf[1] = 0.0
        accf[2] = 0.0
        acci[0] = 0
        acci[1] = 0

    pred_b = pred_ref[0]

    sum_obj_pos = jnp.float32(0.0)
    sum_cls = jnp.float32(0.0)
    sum_loc = jnp.float32(0.0)
    num_pos = jnp.int32(0)
    num_neg = jnp.int32(0)
    negv_list = []

    for a in range(A):
        ax1 = anc_ref[a, 0]
        ay1 = anc_ref[a, 1]
        ax2 = anc_ref[a, 2]
        ay2 = anc_ref[a, 3]
        area_a = (ax2 - ax1) * (ay2 - ay1)

        best = jnp.full((SR, SL), -1.0, jnp.float32)
        mlab = jnp.zeros((SR, SL), jnp.int32)
        bx1 = jnp.zeros((SR, SL), jnp.float32)
        by1 = jnp.zeros((SR, SL), jnp.float32)
        bx2 = jnp.zeros((SR, SL), jnp.float32)
        by2 = jnp.zeros((SR, SL), jnp.float32)
        for g in range(G):
            gx1 = gt_ref[b, g, 0]
            gy1 = gt_ref[b, g, 1]
            gx2 = gt_ref[b, g, 2]
            gy2 = gt_ref[b, g, 3]
            gl = lab_ref[b, g]
            x1 = jnp.maximum(ax1, gx1)
            y1 = jnp.maximum(ay1, gy1)
            x2 = jnp.minimum(ax2, gx2)
            y2 = jnp.minimum(ay2, gy2)
            inter = jnp.clip(x2 - x1, 0.0, None) * jnp.clip(y2 - y1, 0.0, None)
            ag = (gx2 - gx1) * (gy2 - gy1)
            iou = inter / jnp.maximum(area_a + ag - inter, 1e-9)
            upd = iou > best
            best = jnp.where(upd, iou, best)
            mlab = jnp.where(upd, gl, mlab)
            bx1 = jnp.where(upd, gx1, bx1)
            by1 = jnp.where(upd, gy1, by1)
            bx2 = jnp.where(upd, gx2, bx2)
            by2 = jnp.where(upd, gy2, by2)

        pos = best >= POS_T
        neg = best < NEG_T
        posf = pos.astype(jnp.float32)
        num_pos = num_pos + jnp.sum(pos.astype(jnp.int32))
        num_neg = num_neg + jnp.sum(neg.astype(jnp.int32))

        x = pred_b[a * PER + 4]
        bce = jnp.clip(x, 0.0, None) - x * posf + jnp.log1p(jnp.exp(-jnp.abs(x)))
        sum_obj_pos = sum_obj_pos + jnp.sum(jnp.where(pos, bce, 0.0))
        negv_list.append(jnp.where(neg, bce, -1.0))

        logits = pred_b[a * PER + 5: a * PER + 5 + NC]
        m = jnp.max(logits, axis=0)
        lse = m + jnp.log(jnp.sum(jnp.exp(logits - m[None]), axis=0))
        cidx = jax.lax.broadcasted_iota(jnp.int32, (NC, SR, SL), 0)
        picked = jnp.sum(jnp.where(cidx == mlab[None], logits, 0.0), axis=0)
        sum_cls = sum_cls + jnp.sum(jnp.where(pos, lse - picked, 0.0))

        aw = jnp.clip(ax2 - ax1, 1e-6, None)
        ah = jnp.clip(ay2 - ay1, 1e-6, None)
        acx = (ax1 + ax2) * 0.5
        acy = (ay1 + ay2) * 0.5
        gw = jnp.clip(bx2 - bx1, 1e-6, None)
        gh = jnp.clip(by2 - by1, 1e-6, None)
        gcx = (bx1 + bx2) * 0.5
        gcy = (by1 + by2) * 0.5
        tgts = [(gcx - acx) / aw, (gcy - acy) / ah,
                jnp.log(gw / aw), jnp.log(gh / ah)]
        loc_acc = jnp.zeros((SR, SL), jnp.float32)
        for c in range(4):
            d = pred_b[a * PER + c] - tgts[c]
            ad = jnp.abs(d)
            sl = jnp.where(ad < 1.0, 0.5 * d * d, ad - 0.5)
            loc_acc = loc_acc + sl
        sum_loc = sum_loc + jnp.sum(jnp.where(pos, loc_acc, 0.0))

    negv = jnp.stack(negv_list, axis=0)
    k = jnp.where(num_pos > 0, RATIO * num_pos, jnp.minimum(num_neg, 100))
    k = jnp.minimum(k, num_neg)

    iv = jax.lax.bitcast_convert_type(negv, jnp.int32)

    lo_bits = jnp.int32(0)
    for rnd in range(8):
        shift = 28 - 4 * rnd
        njc = 7 if rnd == 0 else 15
        cnts = [jnp.sum((iv >= (lo_bits + (j << shift))).astype(jnp.int32))
                for j in range(1, njc + 1)]
        jstar = cnts[0] * 0
        for c in cnts:
            jstar = jstar + (c >= k).astype(jnp.int32)
        lo_bits = lo_bits + (jstar << shift)
    tval = jnp.max(jnp.where(iv == lo_bits, negv, 0.0))
    cnt_gt = jnp.sum((iv > lo_bits).astype(jnp.int32))
    sum_gt = jnp.sum(jnp.where(iv > lo_bits, negv, 0.0))
    topk = sum_gt + (k - cnt_gt).astype(jnp.float32) * tval
    topk = jnp.where(k > 0, topk, 0.0)

    accf[0] = accf[0] + sum_obj_pos + topk
    accf[1] = accf[1] + sum_cls
    accf[2] = accf[2] + sum_loc
    acci[0] = acci[0] + num_pos
    acci[1] = acci[1] + num_pos + k

    @pl.when(b == B - 1)
    def _final():
        dp = jnp.maximum(acci[0], 1).astype(jnp.float32)
        do = jnp.maximum(acci[1], 1).astype(jnp.float32)
        lo_l = accf[0] / do * W_OBJ
        lc_l = accf[1] / dp * W_CLS
        ll_l = accf[2] / dp * W_LOC
        out_ref[0] = lo_l
        out_ref[1] = lc_l
        out_ref[2] = ll_l
        out_ref[3] = lo_l + lc_l + ll_l


@jax.jit
def kernel(pred, gt_boxes, gt_labels, anchors):
    pred_r = pred.reshape(B, A * PER, SR, SL)
    anc_r = jnp.transpose(anchors.reshape(S, A, 4), (1, 2, 0)).reshape(A, 4, SR, SL)
    gt = gt_boxes.astype(jnp.float32)
    lab = gt_labels.astype(jnp.int32)

    out = pl.pallas_call(
        _loss_kernel,
        grid=(B,),
        in_specs=[
            pl.BlockSpec((1, A * PER, SR, SL), lambda b: (b, 0, 0, 0)),
            pl.BlockSpec((A, 4, SR, SL), lambda b: (0, 0, 0, 0)),
            pl.BlockSpec(memory_space=pltpu.SMEM),
            pl.BlockSpec(memory_space=pltpu.SMEM),
        ],
        out_specs=pl.BlockSpec(memory_space=pltpu.SMEM),
        out_shape=jax.ShapeDtypeStruct((4,), jnp.float32),
        scratch_shapes=[
            pltpu.SMEM((4,), jnp.float32),
            pltpu.SMEM((4,), jnp.int32),
        ],
    )(pred_r, anc_r, gt, lab)
    return out
